# baseline (speedup 1.0000x reference)
import functools

import jax
import jax.numpy as jnp
import numpy as np
from jax import lax
from jax.experimental import pallas as pl
from jax.experimental.pallas import tpu as pltpu
from jax.experimental.pallas import tpu_sc as plsc

NC = 2
NS = 16
LANES = 16
NW = NC * NS

CH = 56
NG = 4
NSB = 2
NE = 8
STRIPE = 624


def _spmm_body(n_nodes, feat, kpw, support_hbm, src_hbm, dst_hbm, aval_hbm,
               out0_hbm, out1_hbm, *scratch):
    esrc = scratch[0:NE]
    edst = scratch[NE:2 * NE]
    eav = scratch[2 * NE:3 * NE]
    gbuf = scratch[3 * NE:3 * NE + NG]
    sbuf = scratch[3 * NE + NG:3 * NE + NG + NSB]
    nrefs = 3 * NE + NG + NSB
    esem = scratch[nrefs:nrefs + NE]
    gsem = scratch[nrefs + NE:nrefs + NE + NG]
    ssem = scratch[nrefs + NE + NG:nrefs + NE + NG + NSB]
    acc = scratch[-1]
    c = lax.axis_index("c")
    s = lax.axis_index("s")
    w = c * NS + s
    jg = feat // LANES
    g0 = gbuf[0]
    k0 = w * kpw

    def eload(k, j):
        pltpu.async_copy(src_hbm.at[pl.ds((k0 + k) * CH, CH)], esrc[j], esem[j])
        pltpu.async_copy(dst_hbm.at[pl.ds((k0 + k) * CH, CH)], edst[j], esem[j])
        pltpu.async_copy(aval_hbm.at[pl.ds((k0 + k) * CH, CH)], eav[j], esem[j])

    def ewait(j):
        pltpu.make_async_copy(src_hbm.at[pl.ds(0, CH)], esrc[j], esem[j]).wait()
        pltpu.make_async_copy(dst_hbm.at[pl.ds(0, CH)], edst[j], esem[j]).wait()
        pltpu.make_async_copy(aval_hbm.at[pl.ds(0, CH)], eav[j], esem[j]).wait()

    zero = jnp.zeros((LANES,), jnp.float32)

    def zrow(r, carry):
        for j in range(jg):
            g0[r, pl.ds(j * LANES, LANES)] = zero
        return carry

    lax.fori_loop(0, CH, zrow, 0)
    row0 = s * STRIPE
    tail0 = NS * STRIPE
    tail_rows = n_nodes - tail0
    for i in range(STRIPE // CH):
        pltpu.sync_copy(g0, acc.at[pl.ds(row0 + i * CH, CH), :])
    rem_rows = STRIPE - (STRIPE // CH) * CH
    if rem_rows:
        pltpu.sync_copy(g0.at[pl.ds(0, rem_rows), :],
                        acc.at[pl.ds(row0 + (STRIPE // CH) * CH, rem_rows), :])
    @pl.when(s == NS - 1)
    def _():
        pltpu.sync_copy(g0.at[pl.ds(0, tail_rows), :],
                        acc.at[pl.ds(tail0, tail_rows), :])
    plsc.subcore_barrier()

    def scale_chunk(av_ref, src_buf, dst_buf):
        @plsc.parallel_loop(0, CH, step=1, unroll=4)
        def _(e):
            af = plsc.load_gather(av_ref, [jnp.full((LANES,), e, jnp.int32)])
            for j in range(jg):
                sl = pl.ds(j * LANES, LANES)
                dst_buf[e, sl] = src_buf[e, sl] * af

    def gwait(b):
        pltpu.make_async_copy(support_hbm.at[esrc[0]], gbuf[b], gsem[b]).wait()

    def swait(sb):
        pltpu.make_async_copy(support_hbm.at[pl.ds(0, CH), :], sbuf[sb],
                              ssem[sb]).wait()

    for j in range(6):
        eload(j, j)
    ewait(0)
    ewait(1)
    pltpu.async_copy(support_hbm.at[esrc[0]], gbuf[0], gsem[0])
    pltpu.async_copy(support_hbm.at[esrc[1]], gbuf[1], gsem[1])
    nr = kpw // NE

    def round_body(r, carry):
        for u in range(NE):
            k = r * NE + u
            gb = u % NG
            sb = u % NSB
            j = u % NE
            gwait(gb)
            if u < 2:
                @pl.when(r > 0)
                def _():
                    swait(sb)
            else:
                swait(sb)
            if u < NE - 2:
                ewait((u + 2) % NE)
                pltpu.async_copy(support_hbm.at[esrc[(u + 2) % NE]],
                                 gbuf[(u + 2) % NG], gsem[(u + 2) % NG])
            else:
                @pl.when(r < nr - 1)
                def _():
                    ewait((u + 2) % NE)
                    pltpu.async_copy(support_hbm.at[esrc[(u + 2) % NE]],
                                     gbuf[(u + 2) % NG], gsem[(u + 2) % NG])
            if u < 2:
                eload(k + 6, (u + 6) % NE)
            else:
                @pl.when(r < nr - 1)
                def _():
                    eload(k + 6, (u + 6) % NE)
            scale_chunk(eav[j], gbuf[gb], sbuf[sb])
            pltpu.async_copy(sbuf[sb], acc.at[edst[j]], ssem[sb], add=True)
        return carry

    lax.fori_loop(0, nr, round_body, 0)
    swait(0)
    swait(1)
    plsc.subcore_barrier()

    crows = 4 * CH
    for i in range(-(-STRIPE // crows)):
        r = row0 + i * crows
        nrow = min(crows, STRIPE - i * crows)
        @pl.when(c == 0)
        def _():
            pltpu.sync_copy(acc.at[pl.ds(r, nrow), :], out0_hbm.at[pl.ds(r, nrow), :])
        @pl.when(c == 1)
        def _():
            pltpu.sync_copy(acc.at[pl.ds(r, nrow), :], out1_hbm.at[pl.ds(r, nrow), :])
    @pl.when(s == NS - 1)
    def _():
        @pl.when(c == 0)
        def _():
            pltpu.sync_copy(acc.at[pl.ds(tail0, tail_rows), :],
                            out0_hbm.at[pl.ds(tail0, tail_rows), :])
        @pl.when(c == 1)
        def _():
            pltpu.sync_copy(acc.at[pl.ds(tail0, tail_rows), :],
                            out1_hbm.at[pl.ds(tail0, tail_rows), :])


def _spmm_partials(support, src_p, dst_p, av_p):
    n_nodes, feat = support.shape
    kpw = src_p.shape[0] // (NW * CH)
    mesh = plsc.VectorSubcoreMesh(core_axis_name="c", subcore_axis_name="s",
                                  num_cores=NC, num_subcores=NS)
    f = pl.kernel(
        functools.partial(_spmm_body, n_nodes, feat, kpw),
        out_type=(jax.ShapeDtypeStruct((n_nodes, feat), jnp.float32),
                  jax.ShapeDtypeStruct((n_nodes, feat), jnp.float32)),
        mesh=mesh,
        compiler_params=pltpu.CompilerParams(needs_layout_passes=False),
        scratch_types=(
            [pltpu.VMEM((CH,), jnp.int32) for _ in range(NE)]
            + [pltpu.VMEM((CH,), jnp.int32) for _ in range(NE)]
            + [pltpu.VMEM((CH,), jnp.float32) for _ in range(NE)]
            + [pltpu.VMEM((CH, feat), jnp.float32) for _ in range(NG)]
            + [pltpu.VMEM((CH, feat), jnp.float32) for _ in range(NSB)]
            + [pltpu.SemaphoreType.DMA for _ in range(NE + NG + NSB)]
            + [pltpu.VMEM_SHARED((n_nodes, feat), jnp.float32)]
        ),
    )
    return f(support, src_p, dst_p, av_p)


def _k1_body(X_ref, Wc_ref, W1_ref, bc_ref, out_ref, wf_s, bf_s):
    @pl.when(pl.program_id(0) == 0)
    def _():
        wf_s[...] = jnp.dot(Wc_ref[...], W1_ref[...],
                            preferred_element_type=jnp.float32)
        bf_s[...] = jnp.dot(bc_ref[...], W1_ref[...],
                            preferred_element_type=jnp.float32)
    out_ref[...] = jnp.dot(X_ref[...], wf_s[...],
                           preferred_element_type=jnp.float32) + bf_s[...]


def _k1(X, W_comp, W1, b_comp):
    n, d = X.shape
    hid = W1.shape[1]
    bm = 1000
    return pl.pallas_call(
        _k1_body,
        grid=(n // bm,),
        in_specs=[
            pl.BlockSpec((bm, d), lambda i: (i, 0)),
            pl.BlockSpec(W_comp.shape, lambda i: (0, 0)),
            pl.BlockSpec(W1.shape, lambda i: (0, 0)),
            pl.BlockSpec((1, W1.shape[0]), lambda i: (0, 0)),
        ],
        out_specs=pl.BlockSpec((bm, hid), lambda i: (i, 0)),
        out_shape=jax.ShapeDtypeStruct((n, hid), jnp.float32),
        scratch_shapes=[
            pltpu.VMEM((d, hid), jnp.float32),
            pltpu.VMEM((1, hid), jnp.float32),
        ],
    )(X, W_comp, W1, b_comp.reshape(1, -1))


def _k2_body(za_ref, zb_ref, b1_ref, W2_ref, out_ref):
    f1 = jnp.maximum(za_ref[...] + zb_ref[...] + b1_ref[...], 0.0)
    out_ref[...] = jnp.dot(f1, W2_ref[...], preferred_element_type=jnp.float32)


def _k2(za, zb, b1, W2):
    n, hid = za.shape
    out_c = W2.shape[1]
    bm = 1000
    return pl.pallas_call(
        _k2_body,
        grid=(n // bm,),
        in_specs=[
            pl.BlockSpec((bm, hid), lambda i: (i, 0)),
            pl.BlockSpec((bm, hid), lambda i: (i, 0)),
            pl.BlockSpec((1, hid), lambda i: (0, 0)),
            pl.BlockSpec(W2.shape, lambda i: (0, 0)),
        ],
        out_specs=pl.BlockSpec((bm, out_c), lambda i: (i, 0)),
        out_shape=jax.ShapeDtypeStruct((n, out_c), jnp.float32),
    )(za, zb, b1.reshape(1, -1), W2)


def _k3_body(x_ref, za_ref, zb_ref, b2_ref, out_ref):
    f2 = za_ref[...] + zb_ref[...] + b2_ref[...]
    acc = lax.dot_general(x_ref[...], f2, (((1,), (1,)), ((), ())),
                          preferred_element_type=jnp.float32)
    out_ref[...] = jax.nn.sigmoid(acc)


def _k3(x, za, zb, b2):
    b, out_c = x.shape
    n = za.shape[0]
    bm = 128
    return pl.pallas_call(
        _k3_body,
        grid=(b // bm,),
        in_specs=[
            pl.BlockSpec((bm, out_c), lambda i: (i, 0)),
            pl.BlockSpec((n, out_c), lambda i: (0, 0)),
            pl.BlockSpec((n, out_c), lambda i: (0, 0)),
            pl.BlockSpec((1, out_c), lambda i: (0, 0)),
        ],
        out_specs=pl.BlockSpec((bm, n), lambda i: (i, 0)),
        out_shape=jax.ShapeDtypeStruct((b, n), jnp.float32),
    )(x, za, zb, b2.reshape(1, -1))


def kernel(x, X, edge_index, a_val, W_comp, b_comp, W1, b1, W2, b2):
    n = X.shape[0]
    src = edge_index[0]
    dst = edge_index[1]
    e = src.shape[0]
    kpw = -(-e // (NW * CH * NE)) * NE
    ep = NW * CH * kpw
    pad = ep - e
    ar = jnp.asarray((np.arange(pad, dtype=np.int32) * 97) % n)
    src_p = jnp.concatenate([src, ar])
    dst_p = jnp.concatenate([dst, ar])
    av_p = jnp.concatenate([a_val, jnp.zeros((pad,), jnp.float32)])
    support1 = _k1(X, W_comp, W1, b_comp)
    z1a, z1b = _spmm_partials(support1, src_p, dst_p, av_p)
    support2 = _k2(z1a, z1b, b1, W2)
    z2a, z2b = _spmm_partials(support2, src_p, dst_p, av_p)
    return _k3(x, z2a, z2b, b2)


# speedup vs baseline: 1.0138x; 1.0138x over previous
import functools

import jax
import jax.numpy as jnp
import numpy as np
from jax import lax
from jax.experimental import pallas as pl
from jax.experimental.pallas import tpu as pltpu
from jax.experimental.pallas import tpu_sc as plsc

NC = 2
NS = 16
LANES = 16
NW = NC * NS

CH = 56
NG = 4
NSB = 2
NE = 8
STRIPE = 624


def _spmm_body(n_nodes, feat, kpw, support_hbm, src_hbm, dst_hbm, aval_hbm,
               out0_hbm, out1_hbm, *scratch):
    esrc = scratch[0:NE]
    edst = scratch[NE:2 * NE]
    eav = scratch[2 * NE:3 * NE]
    gbuf = scratch[3 * NE:3 * NE + NG]
    sbuf = scratch[3 * NE + NG:3 * NE + NG + NSB]
    nrefs = 3 * NE + NG + NSB
    esem = scratch[nrefs:nrefs + NE]
    gsem = scratch[nrefs + NE:nrefs + NE + NG]
    ssem = scratch[nrefs + NE + NG:nrefs + NE + NG + NSB]
    acc = scratch[-1]
    c = lax.axis_index("c")
    s = lax.axis_index("s")
    w = c * NS + s
    jg = feat // LANES
    g0 = gbuf[0]
    k0 = w * kpw

    def eload(k, j):
        pltpu.async_copy(src_hbm.at[pl.ds((k0 + k) * CH, CH)], esrc[j], esem[j])
        pltpu.async_copy(dst_hbm.at[pl.ds((k0 + k) * CH, CH)], edst[j], esem[j])
        pltpu.async_copy(aval_hbm.at[pl.ds((k0 + k) * CH, CH)], eav[j], esem[j])

    def ewait(j):
        pltpu.make_async_copy(src_hbm.at[pl.ds(0, CH)], esrc[j], esem[j]).wait()
        pltpu.make_async_copy(dst_hbm.at[pl.ds(0, CH)], edst[j], esem[j]).wait()
        pltpu.make_async_copy(aval_hbm.at[pl.ds(0, CH)], eav[j], esem[j]).wait()

    zero = jnp.zeros((LANES,), jnp.float32)

    def zrow(r, carry):
        for j in range(jg):
            g0[r, pl.ds(j * LANES, LANES)] = zero
        return carry

    lax.fori_loop(0, CH, zrow, 0)
    row0 = s * STRIPE
    tail0 = NS * STRIPE
    tail_rows = n_nodes - tail0
    for i in range(STRIPE // CH):
        pltpu.sync_copy(g0, acc.at[pl.ds(row0 + i * CH, CH), :])
    rem_rows = STRIPE - (STRIPE // CH) * CH
    if rem_rows:
        pltpu.sync_copy(g0.at[pl.ds(0, rem_rows), :],
                        acc.at[pl.ds(row0 + (STRIPE // CH) * CH, rem_rows), :])
    @pl.when(s == NS - 1)
    def _():
        pltpu.sync_copy(g0.at[pl.ds(0, tail_rows), :],
                        acc.at[pl.ds(tail0, tail_rows), :])
    plsc.subcore_barrier()

    def scale_chunk(av_ref, src_buf, dst_buf):
        @plsc.parallel_loop(0, CH, step=1, unroll=4)
        def _(e):
            af = plsc.load_gather(av_ref, [jnp.full((LANES,), e, jnp.int32)])
            for j in range(jg):
                sl = pl.ds(j * LANES, LANES)
                dst_buf[e, sl] = src_buf[e, sl] * af

    def gwait(b):
        pltpu.make_async_copy(support_hbm.at[esrc[0]], gbuf[b], gsem[b]).wait()

    def swait(sb):
        pltpu.make_async_copy(support_hbm.at[pl.ds(0, CH), :], sbuf[sb],
                              ssem[sb]).wait()

    for j in range(6):
        eload(j, j)
    ewait(0)
    ewait(1)
    pltpu.async_copy(support_hbm.at[esrc[0]], gbuf[0], gsem[0])
    pltpu.async_copy(support_hbm.at[esrc[1]], gbuf[1], gsem[1])
    nr = kpw // NE

    def round_body(r, carry):
        for u in range(NE):
            k = r * NE + u
            gb = u % NG
            sb = u % NSB
            j = u % NE
            gwait(gb)
            if u < 2:
                @pl.when(r > 0)
                def _():
                    swait(sb)
            else:
                swait(sb)
            if u < NE - 2:
                ewait((u + 2) % NE)
                pltpu.async_copy(support_hbm.at[esrc[(u + 2) % NE]],
                                 gbuf[(u + 2) % NG], gsem[(u + 2) % NG])
            else:
                @pl.when(r < nr - 1)
                def _():
                    ewait((u + 2) % NE)
                    pltpu.async_copy(support_hbm.at[esrc[(u + 2) % NE]],
                                     gbuf[(u + 2) % NG], gsem[(u + 2) % NG])
            if u < 2:
                eload(k + 6, (u + 6) % NE)
            else:
                @pl.when(r < nr - 1)
                def _():
                    eload(k + 6, (u + 6) % NE)
            scale_chunk(eav[j], gbuf[gb], sbuf[sb])
            pltpu.async_copy(sbuf[sb], acc.at[edst[j]], ssem[sb], add=True)
        return carry

    lax.fori_loop(0, nr, round_body, 0)
    swait(0)
    swait(1)
    plsc.subcore_barrier()

    crows = 4 * CH
    for i in range(-(-STRIPE // crows)):
        r = row0 + i * crows
        nrow = min(crows, STRIPE - i * crows)
        @pl.when(c == 0)
        def _():
            pltpu.sync_copy(acc.at[pl.ds(r, nrow), :], out0_hbm.at[pl.ds(r, nrow), :])
        @pl.when(c == 1)
        def _():
            pltpu.sync_copy(acc.at[pl.ds(r, nrow), :], out1_hbm.at[pl.ds(r, nrow), :])
    @pl.when(s == NS - 1)
    def _():
        @pl.when(c == 0)
        def _():
            pltpu.sync_copy(acc.at[pl.ds(tail0, tail_rows), :],
                            out0_hbm.at[pl.ds(tail0, tail_rows), :])
        @pl.when(c == 1)
        def _():
            pltpu.sync_copy(acc.at[pl.ds(tail0, tail_rows), :],
                            out1_hbm.at[pl.ds(tail0, tail_rows), :])


def _spmm_partials(support, src_p, dst_p, av_p):
    n_nodes, feat = support.shape
    kpw = src_p.shape[0] // (NW * CH)
    mesh = plsc.VectorSubcoreMesh(core_axis_name="c", subcore_axis_name="s",
                                  num_cores=NC, num_subcores=NS)
    f = pl.kernel(
        functools.partial(_spmm_body, n_nodes, feat, kpw),
        out_type=(jax.ShapeDtypeStruct((n_nodes, feat), jnp.float32),
                  jax.ShapeDtypeStruct((n_nodes, feat), jnp.float32)),
        mesh=mesh,
        compiler_params=pltpu.CompilerParams(needs_layout_passes=False),
        scratch_types=(
            [pltpu.VMEM((CH,), jnp.int32) for _ in range(NE)]
            + [pltpu.VMEM((CH,), jnp.int32) for _ in range(NE)]
            + [pltpu.VMEM((CH,), jnp.float32) for _ in range(NE)]
            + [pltpu.VMEM((CH, feat), jnp.float32) for _ in range(NG)]
            + [pltpu.VMEM((CH, feat), jnp.float32) for _ in range(NSB)]
            + [pltpu.SemaphoreType.DMA for _ in range(NE + NG + NSB)]
            + [pltpu.VMEM_SHARED((n_nodes, feat), jnp.float32)]
        ),
    )
    return f(support, src_p, dst_p, av_p)


def _k1_body(X_ref, Wc_ref, W1_ref, bc_ref, out_ref, wf_s, bf_s):
    @pl.when(pl.program_id(0) == 0)
    def _():
        wf_s[...] = jnp.dot(Wc_ref[...], W1_ref[...],
                            preferred_element_type=jnp.float32)
        bf_s[...] = jnp.dot(bc_ref[...], W1_ref[...],
                            preferred_element_type=jnp.float32)
    out_ref[...] = jnp.dot(X_ref[...], wf_s[...],
                           preferred_element_type=jnp.float32) + bf_s[...]


def _k1(X, W_comp, W1, b_comp):
    n, d = X.shape
    hid = W1.shape[1]
    bm = 2000
    return pl.pallas_call(
        _k1_body,
        grid=(n // bm,),
        in_specs=[
            pl.BlockSpec((bm, d), lambda i: (i, 0)),
            pl.BlockSpec(W_comp.shape, lambda i: (0, 0)),
            pl.BlockSpec(W1.shape, lambda i: (0, 0)),
            pl.BlockSpec((1, W1.shape[0]), lambda i: (0, 0)),
        ],
        out_specs=pl.BlockSpec((bm, hid), lambda i: (i, 0)),
        out_shape=jax.ShapeDtypeStruct((n, hid), jnp.float32),
        scratch_shapes=[
            pltpu.VMEM((d, hid), jnp.float32),
            pltpu.VMEM((1, hid), jnp.float32),
        ],
    )(X, W_comp, W1, b_comp.reshape(1, -1))


def _k2_body(za_ref, zb_ref, b1_ref, W2_ref, out_ref):
    f1 = jnp.maximum(za_ref[...] + zb_ref[...] + b1_ref[...], 0.0)
    out_ref[...] = jnp.dot(f1, W2_ref[...], preferred_element_type=jnp.float32)


def _k2(za, zb, b1, W2):
    n, hid = za.shape
    out_c = W2.shape[1]
    bm = 2000
    return pl.pallas_call(
        _k2_body,
        grid=(n // bm,),
        in_specs=[
            pl.BlockSpec((bm, hid), lambda i: (i, 0)),
            pl.BlockSpec((bm, hid), lambda i: (i, 0)),
            pl.BlockSpec((1, hid), lambda i: (0, 0)),
            pl.BlockSpec(W2.shape, lambda i: (0, 0)),
        ],
        out_specs=pl.BlockSpec((bm, out_c), lambda i: (i, 0)),
        out_shape=jax.ShapeDtypeStruct((n, out_c), jnp.float32),
    )(za, zb, b1.reshape(1, -1), W2)


def _k3_body(x_ref, za_ref, zb_ref, b2_ref, out_ref):
    f2 = za_ref[...] + zb_ref[...] + b2_ref[...]
    acc = lax.dot_general(x_ref[...], f2, (((1,), (1,)), ((), ())),
                          preferred_element_type=jnp.float32)
    out_ref[...] = jax.nn.sigmoid(acc)


def _k3(x, za, zb, b2):
    b, out_c = x.shape
    n = za.shape[0]
    bm = 256
    return pl.pallas_call(
        _k3_body,
        grid=(b // bm,),
        in_specs=[
            pl.BlockSpec((bm, out_c), lambda i: (i, 0)),
            pl.BlockSpec((n, out_c), lambda i: (0, 0)),
            pl.BlockSpec((n, out_c), lambda i: (0, 0)),
            pl.BlockSpec((1, out_c), lambda i: (0, 0)),
        ],
        out_specs=pl.BlockSpec((bm, n), lambda i: (i, 0)),
        out_shape=jax.ShapeDtypeStruct((b, n), jnp.float32),
    )(x, za, zb, b2.reshape(1, -1))


def kernel(x, X, edge_index, a_val, W_comp, b_comp, W1, b1, W2, b2):
    n = X.shape[0]
    src = edge_index[0]
    dst = edge_index[1]
    e = src.shape[0]
    kpw = -(-e // (NW * CH * NE)) * NE
    ep = NW * CH * kpw
    pad = ep - e
    ar = jnp.asarray((np.arange(pad, dtype=np.int32) * 97) % n)
    src_p = jnp.concatenate([src, ar])
    dst_p = jnp.concatenate([dst, ar])
    av_p = jnp.concatenate([a_val, jnp.zeros((pad,), jnp.float32)])
    support1 = _k1(X, W_comp, W1, b_comp)
    z1a, z1b = _spmm_partials(support1, src_p, dst_p, av_p)
    support2 = _k2(z1a, z1b, b1, W2)
    z2a, z2b = _spmm_partials(support2, src_p, dst_p, av_p)
    return _k3(x, z2a, z2b, b2)
